# baseline (speedup 1.0000x reference)
import functools

import jax
import jax.numpy as jnp
from jax import lax
from jax.experimental import pallas as pl
from jax.experimental.pallas import tpu as pltpu
from jax.experimental.pallas import tpu_sc as plsc

N = 10000
D = 128
NC = 2
NS = 16
NW = NC * NS
CHUNK = 128
K_CHUNKS = 80
E_PAD = NW * K_CHUNKS * CHUNK
N_PAD = 10112
ROWS_PER_TILE = N_PAD // NS
DUMMY_DST = N
NBUF = 2
PRIME = 1
GRP = 8
NSB = K_CHUNKS // GRP


def _make_edge_accum(with_counts):
  mesh = plsc.VectorSubcoreMesh(core_axis_name="c", subcore_axis_name="s")
  out_type = [jax.ShapeDtypeStruct((NC, N_PAD, D), jnp.float32)]
  if with_counts:
    out_type.append(jax.ShapeDtypeStruct((NC * N_PAD,), jnp.float32))
  scratch = [
      pltpu.VMEM((2, GRP, CHUNK), jnp.int32),
      pltpu.VMEM((2, GRP, CHUNK), jnp.int32),
      pltpu.VMEM((NBUF, CHUNK, D), jnp.float32),
      pltpu.VMEM((CHUNK,), jnp.float32),
      pltpu.VMEM((ROWS_PER_TILE,), jnp.float32),
      pltpu.VMEM_SHARED((N_PAD, D), jnp.float32),
      pltpu.VMEM_SHARED((N_PAD,), jnp.float32),
  ] + [pltpu.SemaphoreType.DMA] * (2 * NBUF + 1)

  def body(src_hbm, dst_hbm, h_hbm, z2_hbm, z1_hbm, ones_hbm,
           *refs):
    if with_counts:
      acc_out, cnt_out = refs[0], refs[1]
      scratches = refs[2:]
    else:
      acc_out = refs[0]
      scratches = refs[1:]
    (src_w, dst_w, rows_v, ones_v, bounce_v, acc_sh, cnt_sh,
     *sems) = scratches
    gsem = sems[:NBUF]
    ssem = sems[NBUF:2 * NBUF]
    isem = sems[2 * NBUF]

    cid = lax.axis_index("c")
    sid = lax.axis_index("s")
    wid = sid * NC + cid

    rbase = sid * ROWS_PER_TILE
    pltpu.sync_copy(z2_hbm.at[pl.ds(rbase, ROWS_PER_TILE), :],
                    acc_sh.at[pl.ds(rbase, ROWS_PER_TILE), :])
    if with_counts:
      pltpu.sync_copy(z1_hbm.at[pl.ds(rbase, ROWS_PER_TILE)], bounce_v)
      pltpu.sync_copy(bounce_v, cnt_sh.at[pl.ds(rbase, ROWS_PER_TILE)])
      pltpu.sync_copy(ones_hbm, ones_v)

    ibase = wid * K_CHUNKS

    def idx_fetch(sb, p):
      row = ibase + sb * GRP
      pltpu.async_copy(src_hbm.at[pl.ds(row, GRP), :], src_w.at[p], isem)
      pltpu.async_copy(dst_hbm.at[pl.ds(row, GRP), :], dst_w.at[p], isem)

    def idx_wait():
      pltpu.make_async_copy(src_hbm.at[pl.ds(ibase, GRP), :], src_w.at[0],
                            isem).wait()
      pltpu.make_async_copy(dst_hbm.at[pl.ds(ibase, GRP), :], dst_w.at[0],
                            isem).wait()

    idx_fetch(0, 0)
    idx_wait()

    plsc.subcore_barrier()

    def gather_start(p, r, b):
      pltpu.async_copy(h_hbm.at[src_w.at[p, r]], rows_v.at[b], gsem[b])

    def gather_wait(b):
      pltpu.make_async_copy(h_hbm.at[src_w.at[0, 0]], rows_v.at[b],
                            gsem[b]).wait()

    def scatter_start(p, r, b):
      pltpu.async_copy(rows_v.at[b], acc_sh.at[dst_w.at[p, r]], ssem[b],
                       add=True)
      if with_counts:
        pltpu.async_copy(ones_v, cnt_sh.at[dst_w.at[p, r]], ssem[b],
                         add=True)

    def scatter_wait(b):
      pltpu.make_async_copy(rows_v.at[b], acc_sh.at[dst_w.at[0, 0]],
                            ssem[b]).wait()
      if with_counts:
        pltpu.make_async_copy(ones_v, cnt_sh.at[dst_w.at[0, 0]],
                              ssem[b]).wait()

    def sb_steps(sb, p, first=False, last=False):
      for r in range(GRP):
        b = r % NBUF
        b2 = (b + PRIME) % NBUF
        if not (first and r < NBUF - PRIME):
          scatter_wait(b2)
        if r == 2 and not last:
          idx_fetch(sb + 1, 1 - p)
        if r == GRP - PRIME and not last:
          idx_wait()
        if not (last and r >= GRP - PRIME):
          if r < GRP - PRIME:
            gather_start(p, r + PRIME, b2)
          else:
            gather_start(1 - p, r - (GRP - PRIME), b2)
        gather_wait(b)
        scatter_start(p, r, b)

    for b in range(PRIME):
      gather_start(0, b, b)
    sb_steps(0, 0, first=True)
    sb_steps(1, 1)

    def pair(g, carry):
      sb = 2 * g
      sb_steps(sb, 0)
      sb_steps(sb + 1, 1)
      return carry

    lax.fori_loop(1, NSB // 2 - 1, pair, 0)

    sb_steps(NSB - 2, 0)
    sb_steps(NSB - 1, 1, last=True)
    for k in range(K_CHUNKS - (NBUF - PRIME), K_CHUNKS):
      scatter_wait(k % NBUF)

    plsc.subcore_barrier()

    pltpu.sync_copy(acc_sh.at[pl.ds(rbase, ROWS_PER_TILE), :],
                    acc_out.at[cid, pl.ds(rbase, ROWS_PER_TILE), :])
    if with_counts:
      pltpu.sync_copy(cnt_sh.at[pl.ds(rbase, ROWS_PER_TILE)], bounce_v)
      pltpu.sync_copy(bounce_v,
                      cnt_out.at[pl.ds(cid * N_PAD + rbase, ROWS_PER_TILE)])

  return pl.kernel(body, out_type=out_type, mesh=mesh,
                   scratch_types=scratch)


def _matmul_wt(x, w):
  def body(x_ref, w_ref, o_ref):
    o_ref[...] = lax.dot_general(
        x_ref[...], w_ref[...], (((1,), (1,)), ((), ())),
        preferred_element_type=jnp.float32)
  return pl.pallas_call(
      body,
      out_shape=jax.ShapeDtypeStruct((x.shape[0], w.shape[0]), jnp.float32),
  )(x, w)


_GRID_R = 2000


def _fused_layer1(h1, acc, cnt, feats, w2):
  def body(h_ref, a_ref, c_ref, f_ref, w_ref, dinv_ref, avg_ref, h2_ref):
    cnt_rows = c_ref[...]
    dinv = 1.0 / (1.0 + cnt_rows[0] + cnt_rows[1])
    dinv_ref[...] = dinv
    y = dinv * (h_ref[...] + a_ref[0] + a_ref[1])
    nrm = jnp.maximum(jnp.sqrt(jnp.sum(y * y, axis=1, keepdims=True)), 1e-12)
    avg_ref[...] = f_ref[...] * (1.0 / 3.0) + (y / nrm) * (1.0 / 3.0)
    h2_ref[...] = lax.dot_general(
        y, w_ref[...], (((1,), (1,)), ((), ())),
        preferred_element_type=jnp.float32)

  grid = N // _GRID_R
  return pl.pallas_call(
      body,
      grid=(grid,),
      in_specs=[
          pl.BlockSpec((_GRID_R, D), lambda i: (i, 0)),
          pl.BlockSpec((NC, _GRID_R, D), lambda i: (0, i, 0)),
          pl.BlockSpec((NC, _GRID_R, 1), lambda i: (0, i, 0)),
          pl.BlockSpec((_GRID_R, D), lambda i: (i, 0)),
          pl.BlockSpec((D, D), lambda i: (0, 0)),
      ],
      out_specs=[
          pl.BlockSpec((_GRID_R, 1), lambda i: (i, 0)),
          pl.BlockSpec((_GRID_R, D), lambda i: (i, 0)),
          pl.BlockSpec((_GRID_R, D), lambda i: (i, 0)),
      ],
      out_shape=[
          jax.ShapeDtypeStruct((N, 1), jnp.float32),
          jax.ShapeDtypeStruct((N, D), jnp.float32),
          jax.ShapeDtypeStruct((N, D), jnp.float32),
      ],
  )(h1, acc, cnt, feats, w2)


def _fused_layer2(h2, acc, dinv, avg1):
  def body(h_ref, a_ref, d_ref, avg1_ref, out_ref):
    y = d_ref[...] * (h_ref[...] + a_ref[0] + a_ref[1])
    nrm = jnp.maximum(jnp.sqrt(jnp.sum(y * y, axis=1, keepdims=True)), 1e-12)
    out_ref[...] = avg1_ref[...] + (y / nrm) * (1.0 / 3.0)

  grid = N // _GRID_R
  return pl.pallas_call(
      body,
      grid=(grid,),
      in_specs=[
          pl.BlockSpec((_GRID_R, D), lambda i: (i, 0)),
          pl.BlockSpec((NC, _GRID_R, D), lambda i: (0, i, 0)),
          pl.BlockSpec((_GRID_R, 1), lambda i: (i, 0)),
          pl.BlockSpec((_GRID_R, D), lambda i: (i, 0)),
      ],
      out_specs=pl.BlockSpec((_GRID_R, D), lambda i: (i, 0)),
      out_shape=jax.ShapeDtypeStruct((N, D), jnp.float32),
  )(h2, acc, dinv, avg1)


_edge_accum_counts = _make_edge_accum(with_counts=True)
_edge_accum = _make_edge_accum(with_counts=False)


def kernel(features, edge_index, W1, W2):
  src = edge_index[0].astype(jnp.int32)
  dst = edge_index[1].astype(jnp.int32)
  pad = E_PAD - src.shape[0]
  src_p = jnp.concatenate([src, jnp.zeros((pad,), jnp.int32)])
  dst_p = jnp.concatenate([dst, jnp.full((pad,), DUMMY_DST, jnp.int32)])
  src2 = src_p.reshape(NW * K_CHUNKS, CHUNK)
  dst2 = dst_p.reshape(NW * K_CHUNKS, CHUNK)
  z2 = jnp.zeros((N_PAD, D), jnp.float32)
  z1 = jnp.zeros((N_PAD,), jnp.float32)
  ones = jnp.ones((CHUNK,), jnp.float32)

  h1 = _matmul_wt(features, W1)
  acc1, cnt = _edge_accum_counts(src2, dst2, h1, z2, z1, ones)
  acc1 = acc1[:, :N, :]
  cnt3 = cnt.reshape(NC, N_PAD)[:, :N, None]
  dinv, avg1, h2 = _fused_layer1(h1, acc1, cnt3, features, W2)
  [acc2] = _edge_accum(src2, dst2, h2, z2, z1, ones)
  acc2 = acc2[:, :N, :]
  return _fused_layer2(h2, acc2, dinv, avg1)


# speedup vs baseline: 1.0524x; 1.0524x over previous
import functools

import jax
import jax.numpy as jnp
from jax import lax
from jax.experimental import pallas as pl
from jax.experimental.pallas import tpu as pltpu
from jax.experimental.pallas import tpu_sc as plsc

N = 10000
D = 128
NC = 2
NS = 16
NW = NC * NS
CHUNK = 32
K_CHUNKS = 320
E_PAD = NW * K_CHUNKS * CHUNK
N_PAD = 10112
ROWS_PER_TILE = N_PAD // NS
DUMMY_DST = N
NBUF = 8
PRIME = 6
GRP = 16
NSB = K_CHUNKS // GRP


def _make_edge_accum(with_counts):
  mesh = plsc.VectorSubcoreMesh(core_axis_name="c", subcore_axis_name="s")
  out_type = [jax.ShapeDtypeStruct((NC, N_PAD, D), jnp.float32)]
  if with_counts:
    out_type.append(jax.ShapeDtypeStruct((NC * N_PAD,), jnp.float32))
  scratch = [
      pltpu.VMEM((2, GRP, CHUNK), jnp.int32),
      pltpu.VMEM((2, GRP, CHUNK), jnp.int32),
      pltpu.VMEM((NBUF, CHUNK, D), jnp.float32),
      pltpu.VMEM((CHUNK,), jnp.float32),
      pltpu.VMEM((ROWS_PER_TILE,), jnp.float32),
      pltpu.VMEM_SHARED((N_PAD, D), jnp.float32),
      pltpu.VMEM_SHARED((N_PAD,), jnp.float32),
  ] + [pltpu.SemaphoreType.DMA] * (2 * NBUF + 1)

  def body(src_hbm, dst_hbm, h_hbm, z2_hbm, z1_hbm, ones_hbm,
           *refs):
    if with_counts:
      acc_out, cnt_out = refs[0], refs[1]
      scratches = refs[2:]
    else:
      acc_out = refs[0]
      scratches = refs[1:]
    (src_w, dst_w, rows_v, ones_v, bounce_v, acc_sh, cnt_sh,
     *sems) = scratches
    gsem = sems[:NBUF]
    ssem = sems[NBUF:2 * NBUF]
    isem = sems[2 * NBUF]

    cid = lax.axis_index("c")
    sid = lax.axis_index("s")
    wid = sid * NC + cid

    rbase = sid * ROWS_PER_TILE
    pltpu.sync_copy(z2_hbm.at[pl.ds(rbase, ROWS_PER_TILE), :],
                    acc_sh.at[pl.ds(rbase, ROWS_PER_TILE), :])
    if with_counts:
      pltpu.sync_copy(z1_hbm.at[pl.ds(rbase, ROWS_PER_TILE)], bounce_v)
      pltpu.sync_copy(bounce_v, cnt_sh.at[pl.ds(rbase, ROWS_PER_TILE)])
      pltpu.sync_copy(ones_hbm, ones_v)

    ibase = wid * K_CHUNKS

    def idx_fetch(sb, p):
      row = ibase + sb * GRP
      pltpu.async_copy(src_hbm.at[pl.ds(row, GRP), :], src_w.at[p], isem)
      pltpu.async_copy(dst_hbm.at[pl.ds(row, GRP), :], dst_w.at[p], isem)

    def idx_wait():
      pltpu.make_async_copy(src_hbm.at[pl.ds(ibase, GRP), :], src_w.at[0],
                            isem).wait()
      pltpu.make_async_copy(dst_hbm.at[pl.ds(ibase, GRP), :], dst_w.at[0],
                            isem).wait()

    idx_fetch(0, 0)
    idx_wait()

    plsc.subcore_barrier()

    def gather_start(p, r, b):
      pltpu.async_copy(h_hbm.at[src_w.at[p, r]], rows_v.at[b], gsem[b])

    def gather_wait(b):
      pltpu.make_async_copy(h_hbm.at[src_w.at[0, 0]], rows_v.at[b],
                            gsem[b]).wait()

    def scatter_start(p, r, b):
      pltpu.async_copy(rows_v.at[b], acc_sh.at[dst_w.at[p, r]], ssem[b],
                       add=True)
      if with_counts:
        pltpu.async_copy(ones_v, cnt_sh.at[dst_w.at[p, r]], ssem[b],
                         add=True)

    def scatter_wait(b):
      pltpu.make_async_copy(rows_v.at[b], acc_sh.at[dst_w.at[0, 0]],
                            ssem[b]).wait()
      if with_counts:
        pltpu.make_async_copy(ones_v, cnt_sh.at[dst_w.at[0, 0]],
                              ssem[b]).wait()

    def sb_steps(sb, p, first=False, last=False):
      for r in range(GRP):
        b = r % NBUF
        b2 = (b + PRIME) % NBUF
        if not (first and r < NBUF - PRIME):
          scatter_wait(b2)
        if r == 2 and not last:
          idx_fetch(sb + 1, 1 - p)
        if r == GRP - PRIME and not last:
          idx_wait()
        if not (last and r >= GRP - PRIME):
          if r < GRP - PRIME:
            gather_start(p, r + PRIME, b2)
          else:
            gather_start(1 - p, r - (GRP - PRIME), b2)
        gather_wait(b)
        scatter_start(p, r, b)

    for b in range(PRIME):
      gather_start(0, b, b)
    sb_steps(0, 0, first=True)
    sb_steps(1, 1)

    def pair(g, carry):
      sb = 2 * g
      sb_steps(sb, 0)
      sb_steps(sb + 1, 1)
      return carry

    lax.fori_loop(1, NSB // 2 - 1, pair, 0)

    sb_steps(NSB - 2, 0)
    sb_steps(NSB - 1, 1, last=True)
    for k in range(K_CHUNKS - (NBUF - PRIME), K_CHUNKS):
      scatter_wait(k % NBUF)

    plsc.subcore_barrier()

    pltpu.sync_copy(acc_sh.at[pl.ds(rbase, ROWS_PER_TILE), :],
                    acc_out.at[cid, pl.ds(rbase, ROWS_PER_TILE), :])
    if with_counts:
      pltpu.sync_copy(cnt_sh.at[pl.ds(rbase, ROWS_PER_TILE)], bounce_v)
      pltpu.sync_copy(bounce_v,
                      cnt_out.at[pl.ds(cid * N_PAD + rbase, ROWS_PER_TILE)])

  return pl.kernel(body, out_type=out_type, mesh=mesh,
                   scratch_types=scratch)


def _matmul_wt(x, w):
  def body(x_ref, w_ref, o_ref):
    o_ref[...] = lax.dot_general(
        x_ref[...], w_ref[...], (((1,), (1,)), ((), ())),
        preferred_element_type=jnp.float32)
  return pl.pallas_call(
      body,
      out_shape=jax.ShapeDtypeStruct((x.shape[0], w.shape[0]), jnp.float32),
  )(x, w)


_GRID_R = 2000


def _fused_layer1(h1, acc, cnt, feats, w2):
  def body(h_ref, a_ref, c_ref, f_ref, w_ref, dinv_ref, avg_ref, h2_ref):
    cnt_rows = c_ref[...]
    dinv = 1.0 / (1.0 + cnt_rows[0] + cnt_rows[1])
    dinv_ref[...] = dinv
    y = dinv * (h_ref[...] + a_ref[0] + a_ref[1])
    nrm = jnp.maximum(jnp.sqrt(jnp.sum(y * y, axis=1, keepdims=True)), 1e-12)
    avg_ref[...] = f_ref[...] * (1.0 / 3.0) + (y / nrm) * (1.0 / 3.0)
    h2_ref[...] = lax.dot_general(
        y, w_ref[...], (((1,), (1,)), ((), ())),
        preferred_element_type=jnp.float32)

  grid = N // _GRID_R
  return pl.pallas_call(
      body,
      grid=(grid,),
      in_specs=[
          pl.BlockSpec((_GRID_R, D), lambda i: (i, 0)),
          pl.BlockSpec((NC, _GRID_R, D), lambda i: (0, i, 0)),
          pl.BlockSpec((NC, _GRID_R, 1), lambda i: (0, i, 0)),
          pl.BlockSpec((_GRID_R, D), lambda i: (i, 0)),
          pl.BlockSpec((D, D), lambda i: (0, 0)),
      ],
      out_specs=[
          pl.BlockSpec((_GRID_R, 1), lambda i: (i, 0)),
          pl.BlockSpec((_GRID_R, D), lambda i: (i, 0)),
          pl.BlockSpec((_GRID_R, D), lambda i: (i, 0)),
      ],
      out_shape=[
          jax.ShapeDtypeStruct((N, 1), jnp.float32),
          jax.ShapeDtypeStruct((N, D), jnp.float32),
          jax.ShapeDtypeStruct((N, D), jnp.float32),
      ],
  )(h1, acc, cnt, feats, w2)


def _fused_layer2(h2, acc, dinv, avg1):
  def body(h_ref, a_ref, d_ref, avg1_ref, out_ref):
    y = d_ref[...] * (h_ref[...] + a_ref[0] + a_ref[1])
    nrm = jnp.maximum(jnp.sqrt(jnp.sum(y * y, axis=1, keepdims=True)), 1e-12)
    out_ref[...] = avg1_ref[...] + (y / nrm) * (1.0 / 3.0)

  grid = N // _GRID_R
  return pl.pallas_call(
      body,
      grid=(grid,),
      in_specs=[
          pl.BlockSpec((_GRID_R, D), lambda i: (i, 0)),
          pl.BlockSpec((NC, _GRID_R, D), lambda i: (0, i, 0)),
          pl.BlockSpec((_GRID_R, 1), lambda i: (i, 0)),
          pl.BlockSpec((_GRID_R, D), lambda i: (i, 0)),
      ],
      out_specs=pl.BlockSpec((_GRID_R, D), lambda i: (i, 0)),
      out_shape=jax.ShapeDtypeStruct((N, D), jnp.float32),
  )(h2, acc, dinv, avg1)


_edge_accum_counts = _make_edge_accum(with_counts=True)
_edge_accum = _make_edge_accum(with_counts=False)


def kernel(features, edge_index, W1, W2):
  src = edge_index[0].astype(jnp.int32)
  dst = edge_index[1].astype(jnp.int32)
  pad = E_PAD - src.shape[0]
  src_p = jnp.concatenate([src, jnp.zeros((pad,), jnp.int32)])
  dst_p = jnp.concatenate([dst, jnp.full((pad,), DUMMY_DST, jnp.int32)])
  src2 = src_p.reshape(NW * K_CHUNKS, CHUNK)
  dst2 = dst_p.reshape(NW * K_CHUNKS, CHUNK)
  z2 = jnp.zeros((N_PAD, D), jnp.float32)
  z1 = jnp.zeros((N_PAD,), jnp.float32)
  ones = jnp.ones((CHUNK,), jnp.float32)

  h1 = _matmul_wt(features, W1)
  acc1, cnt = _edge_accum_counts(src2, dst2, h1, z2, z1, ones)
  acc1 = acc1[:, :N, :]
  cnt3 = cnt.reshape(NC, N_PAD)[:, :N, None]
  dinv, avg1, h2 = _fused_layer1(h1, acc1, cnt3, features, W2)
  [acc2] = _edge_accum(src2, dst2, h2, z2, z1, ones)
  acc2 = acc2[:, :N, :]
  return _fused_layer2(h2, acc2, dinv, avg1)


# speedup vs baseline: 1.0814x; 1.0276x over previous
import jax
import jax.numpy as jnp
from jax import lax
from jax.experimental import pallas as pl
from jax.experimental.pallas import tpu as pltpu
from jax.experimental.pallas import tpu_sc as plsc

N = 10000
D = 128
NC = 2
NS = 16
NW = NC * NS
CHUNK = 64
K_CHUNKS = 160
E_PAD = NW * K_CHUNKS * CHUNK
N_PAD = 10112
ROWS_PER_TILE = N_PAD // NS
DUMMY_DST = N
NBUF = 4
PRIME = 3
GRP = 8
NSB = K_CHUNKS // GRP


def _make_edge_accum(with_counts):
  mesh = plsc.VectorSubcoreMesh(core_axis_name="c", subcore_axis_name="s")
  out_type = [jax.ShapeDtypeStruct((NC, N_PAD, D), jnp.float32)]
  if with_counts:
    out_type.append(jax.ShapeDtypeStruct((NC * N_PAD,), jnp.float32))
  scratch = [
      pltpu.VMEM((2, GRP, CHUNK), jnp.int32),
      pltpu.VMEM((2, GRP, CHUNK), jnp.int32),
      pltpu.VMEM((NBUF, CHUNK, D), jnp.float32),
      pltpu.VMEM((CHUNK,), jnp.float32),
      pltpu.VMEM((ROWS_PER_TILE,), jnp.float32),
      pltpu.VMEM_SHARED((N_PAD, D), jnp.float32),
      pltpu.VMEM_SHARED((N_PAD,), jnp.float32),
  ] + [pltpu.SemaphoreType.DMA] * (2 * NBUF + 1)

  def body(src_hbm, dst_hbm, h_hbm, z2_hbm, z1_hbm, ones_hbm, *refs):
    if with_counts:
      acc_out, cnt_out = refs[0], refs[1]
      scratches = refs[2:]
    else:
      acc_out = refs[0]
      scratches = refs[1:]
    (src_w, dst_w, rows_v, ones_v, bounce_v, acc_sh, cnt_sh,
     *sems) = scratches
    gsem = sems[:NBUF]
    ssem = sems[NBUF:2 * NBUF]
    isem = sems[2 * NBUF]

    cid = lax.axis_index("c")
    sid = lax.axis_index("s")
    wid = sid * NC + cid

    rbase = sid * ROWS_PER_TILE
    pltpu.sync_copy(z2_hbm.at[pl.ds(rbase, ROWS_PER_TILE), :],
                    acc_sh.at[pl.ds(rbase, ROWS_PER_TILE), :])
    if with_counts:
      pltpu.sync_copy(z1_hbm.at[pl.ds(rbase, ROWS_PER_TILE)], bounce_v)
      pltpu.sync_copy(bounce_v, cnt_sh.at[pl.ds(rbase, ROWS_PER_TILE)])
      pltpu.sync_copy(ones_hbm, ones_v)

    ibase = wid * K_CHUNKS

    def idx_fetch(sb, p):
      row = ibase + sb * GRP
      pltpu.async_copy(src_hbm.at[pl.ds(row, GRP), :], src_w.at[p], isem)
      pltpu.async_copy(dst_hbm.at[pl.ds(row, GRP), :], dst_w.at[p], isem)

    def idx_wait():
      pltpu.make_async_copy(src_hbm.at[pl.ds(ibase, GRP), :], src_w.at[0],
                            isem).wait()
      pltpu.make_async_copy(dst_hbm.at[pl.ds(ibase, GRP), :], dst_w.at[0],
                            isem).wait()

    idx_fetch(0, 0)
    idx_wait()

    plsc.subcore_barrier()

    def gather_start(p, r, b):
      pltpu.async_copy(h_hbm.at[src_w.at[p, r]], rows_v.at[b], gsem[b])

    def gather_wait(b):
      pltpu.make_async_copy(h_hbm.at[src_w.at[0, 0]], rows_v.at[b],
                            gsem[b]).wait()

    def scatter_start(p, r, b):
      pltpu.async_copy(rows_v.at[b], acc_sh.at[dst_w.at[p, r]], ssem[b],
                       add=True)
      if with_counts:
        pltpu.async_copy(ones_v, cnt_sh.at[dst_w.at[p, r]], ssem[b],
                         add=True)

    def scatter_wait(b):
      pltpu.make_async_copy(rows_v.at[b], acc_sh.at[dst_w.at[0, 0]],
                            ssem[b]).wait()
      if with_counts:
        pltpu.make_async_copy(ones_v, cnt_sh.at[dst_w.at[0, 0]],
                              ssem[b]).wait()

    def sb_steps(sb, p, first=False, last=False):
      for r in range(GRP):
        b = r % NBUF
        b2 = (b + PRIME) % NBUF
        if not (first and r < NBUF - PRIME):
          scatter_wait(b2)
        if r == 2 and not last:
          idx_fetch(sb + 1, 1 - p)
        if r == GRP - PRIME and not last:
          idx_wait()
        if not (last and r >= GRP - PRIME):
          if r < GRP - PRIME:
            gather_start(p, r + PRIME, b2)
          else:
            gather_start(1 - p, r - (GRP - PRIME), b2)
        gather_wait(b)
        scatter_start(p, r, b)

    for b in range(PRIME):
      gather_start(0, b, b)
    sb_steps(0, 0, first=True)
    sb_steps(1, 1)

    def pair(g, carry):
      sb = 2 * g
      sb_steps(sb, 0)
      sb_steps(sb + 1, 1)
      return carry

    lax.fori_loop(1, NSB // 2 - 1, pair, 0)

    sb_steps(NSB - 2, 0)
    sb_steps(NSB - 1, 1, last=True)
    for k in range(K_CHUNKS - (NBUF - PRIME), K_CHUNKS):
      scatter_wait(k % NBUF)

    plsc.subcore_barrier()

    pltpu.sync_copy(acc_sh.at[pl.ds(rbase, ROWS_PER_TILE), :],
                    acc_out.at[cid, pl.ds(rbase, ROWS_PER_TILE), :])
    if with_counts:
      pltpu.sync_copy(cnt_sh.at[pl.ds(rbase, ROWS_PER_TILE)], bounce_v)
      pltpu.sync_copy(bounce_v,
                      cnt_out.at[pl.ds(cid * N_PAD + rbase, ROWS_PER_TILE)])

  return pl.kernel(body, out_type=out_type, mesh=mesh,
                   scratch_types=scratch)


def _matmul_wt(x, w):
  def body(x_ref, w_ref, o_ref):
    o_ref[...] = lax.dot_general(
        x_ref[...], w_ref[...], (((1,), (1,)), ((), ())),
        preferred_element_type=jnp.float32)
  return pl.pallas_call(
      body,
      out_shape=jax.ShapeDtypeStruct((x.shape[0], w.shape[0]), jnp.float32),
  )(x, w)


_GRID_R = 2000


def _fused_layer1(h1, acc, cnt, feats, w2):
  def body(h_ref, a_ref, c_ref, f_ref, w_ref, dinv_ref, avg_ref, h2_ref):
    cnt_rows = c_ref[...]
    dinv = 1.0 / (1.0 + cnt_rows[0] + cnt_rows[1])
    dinv_ref[...] = dinv
    y = dinv * (h_ref[...] + a_ref[0] + a_ref[1])
    nrm = jnp.maximum(jnp.sqrt(jnp.sum(y * y, axis=1, keepdims=True)), 1e-12)
    avg_ref[...] = f_ref[...] * (1.0 / 3.0) + (y / nrm) * (1.0 / 3.0)
    h2_ref[...] = lax.dot_general(
        y, w_ref[...], (((1,), (1,)), ((), ())),
        preferred_element_type=jnp.float32)

  grid = N // _GRID_R
  return pl.pallas_call(
      body,
      grid=(grid,),
      in_specs=[
          pl.BlockSpec((_GRID_R, D), lambda i: (i, 0)),
          pl.BlockSpec((NC, _GRID_R, D), lambda i: (0, i, 0)),
          pl.BlockSpec((NC, _GRID_R, 1), lambda i: (0, i, 0)),
          pl.BlockSpec((_GRID_R, D), lambda i: (i, 0)),
          pl.BlockSpec((D, D), lambda i: (0, 0)),
      ],
      out_specs=[
          pl.BlockSpec((_GRID_R, 1), lambda i: (i, 0)),
          pl.BlockSpec((_GRID_R, D), lambda i: (i, 0)),
          pl.BlockSpec((_GRID_R, D), lambda i: (i, 0)),
      ],
      out_shape=[
          jax.ShapeDtypeStruct((N, 1), jnp.float32),
          jax.ShapeDtypeStruct((N, D), jnp.float32),
          jax.ShapeDtypeStruct((N, D), jnp.float32),
      ],
  )(h1, acc, cnt, feats, w2)


def _fused_layer2(h2, acc, dinv, avg1):
  def body(h_ref, a_ref, d_ref, avg1_ref, out_ref):
    y = d_ref[...] * (h_ref[...] + a_ref[0] + a_ref[1])
    nrm = jnp.maximum(jnp.sqrt(jnp.sum(y * y, axis=1, keepdims=True)), 1e-12)
    out_ref[...] = avg1_ref[...] + (y / nrm) * (1.0 / 3.0)

  grid = N // _GRID_R
  return pl.pallas_call(
      body,
      grid=(grid,),
      in_specs=[
          pl.BlockSpec((_GRID_R, D), lambda i: (i, 0)),
          pl.BlockSpec((NC, _GRID_R, D), lambda i: (0, i, 0)),
          pl.BlockSpec((_GRID_R, 1), lambda i: (i, 0)),
          pl.BlockSpec((_GRID_R, D), lambda i: (i, 0)),
      ],
      out_specs=pl.BlockSpec((_GRID_R, D), lambda i: (i, 0)),
      out_shape=jax.ShapeDtypeStruct((N, D), jnp.float32),
  )(h2, acc, dinv, avg1)


_edge_accum_counts = _make_edge_accum(with_counts=True)
_edge_accum = _make_edge_accum(with_counts=False)


def kernel(features, edge_index, W1, W2):
  src = edge_index[0].astype(jnp.int32)
  dst = edge_index[1].astype(jnp.int32)
  pad = E_PAD - src.shape[0]
  src_p = jnp.concatenate([src, jnp.zeros((pad,), jnp.int32)])
  dst_p = jnp.concatenate([dst, jnp.full((pad,), DUMMY_DST, jnp.int32)])
  src2 = src_p.reshape(NW * K_CHUNKS, CHUNK)
  dst2 = dst_p.reshape(NW * K_CHUNKS, CHUNK)
  z2 = jnp.zeros((N_PAD, D), jnp.float32)
  z1 = jnp.zeros((N_PAD,), jnp.float32)
  ones = jnp.ones((CHUNK,), jnp.float32)

  h1 = _matmul_wt(features, W1)
  acc1, cnt = _edge_accum_counts(src2, dst2, h1, z2, z1, ones)
  acc1 = acc1[:, :N, :]
  cnt3 = cnt.reshape(NC, N_PAD)[:, :N, None]
  dinv, avg1, h2 = _fused_layer1(h1, acc1, cnt3, features, W2)
  [acc2] = _edge_accum(src2, dst2, h2, z2, z1, ones)
  acc2 = acc2[:, :N, :]
  return _fused_layer2(h2, acc2, dinv, avg1)
